# Initial kernel scaffold; baseline (speedup 1.0000x reference)
import jax
import jax.numpy as jnp
from jax.experimental import pallas as pl


def kernel(x_names, x_types, x_behaviors, edge_index, edge_weight, batch, name_table, type_table, W1, b1, W2, b2, Wc, bc):
    raise NotImplementedError("write your pallas kernel here")


# speedup vs baseline: 9.0165x; 9.0165x over previous
import dataclasses

import jax
import jax.numpy as jnp
from jax import lax
from jax.experimental import pallas as pl
from jax.experimental.pallas import tpu as pltpu
from jax.experimental.pallas import tpu_sc as plsc

N = 10000
E = 320000
HID = 128
NGRAPH = 64
TYPE_V = 64
TYPE_D = 16
NAME_D = 64

NSC = 2
NSUB = 16
LANES = 16
NW = NSC * NSUB

EC = 128
E_PER_SC = E // NSC
NCHUNK_SC = E_PER_SC // EC
RS = 624
TAIL = N - RS * NSUB

NAMC = 80
NAME_CHUNKS = N // NAMC

_mesh = plsc.VectorSubcoreMesh(core_axis_name="c", subcore_axis_name="s")

_sc_params = pltpu.CompilerParams()
_fields = pltpu.CompilerParams.__dataclass_fields__
if "needs_layout_passes" in _fields:
    _sc_params = dataclasses.replace(_sc_params, needs_layout_passes=False)
if "use_tc_tiling_on_sc" in _fields:
    _sc_params = dataclasses.replace(_sc_params, use_tc_tiling_on_sc=False)


def _sc_prep_body(dst_hbm, ew_hbm, names_hbm, table_hbm, z16_hbm,
                  degp_hbm, nfeat_hbm,
                  idx_v, nrow_v, dst_v, ew_v, deg_rows, deg_sh, sem):
    cid = lax.axis_index("c")
    sid = lax.axis_index("s")
    wid = sid * NSC + cid

    @pl.loop(wid, NAME_CHUNKS, step=NW)
    def _(j):
        base = j * NAMC
        pltpu.sync_copy(names_hbm.at[pl.ds(base, NAMC)], idx_v)
        pltpu.async_copy(table_hbm.at[idx_v], nrow_v, sem).wait()
        pltpu.sync_copy(nrow_v, nfeat_hbm.at[pl.ds(base, NAMC)])

    r0 = sid * RS
    pltpu.sync_copy(z16_hbm.at[pl.ds(r0, RS)], deg_sh.at[pl.ds(r0, RS)])

    @pl.when(sid == 0)
    def _():
        pltpu.sync_copy(z16_hbm.at[pl.ds(RS * NSUB, TAIL)],
                        deg_sh.at[pl.ds(RS * NSUB, TAIL)])

    plsc.subcore_barrier()

    @pl.loop(sid, NCHUNK_SC, step=NSUB)
    def _(j):
        base = cid * E_PER_SC + j * EC
        pltpu.sync_copy(dst_hbm.at[pl.ds(base, EC)], dst_v)
        pltpu.sync_copy(ew_hbm.at[pl.ds(base, EC)], ew_v)

        @pl.loop(0, EC)
        def _(e):
            deg_rows[e, :] = plsc.load_gather(
                ew_v, [jnp.full((LANES,), e, jnp.int32)])

        pltpu.sync_copy(deg_rows, deg_sh.at[dst_v], add=True)

    plsc.subcore_barrier()
    pltpu.sync_copy(deg_sh.at[pl.ds(r0, RS)],
                    degp_hbm.at[cid, pl.ds(r0, RS)])

    @pl.when(sid == 0)
    def _():
        pltpu.sync_copy(deg_sh.at[pl.ds(RS * NSUB, TAIL)],
                        degp_hbm.at[cid, pl.ds(RS * NSUB, TAIL)])


_sc_prep = pl.kernel(
    _sc_prep_body,
    out_type=(jax.ShapeDtypeStruct((NSC, N, LANES), jnp.float32),
              jax.ShapeDtypeStruct((N, NAME_D), jnp.float32)),
    mesh=_mesh,
    scratch_types=[
        pltpu.VMEM((NAMC,), jnp.int32),
        pltpu.VMEM((NAMC, NAME_D), jnp.float32),
        pltpu.VMEM((EC,), jnp.int32),
        pltpu.VMEM((EC,), jnp.float32),
        pltpu.VMEM((EC, LANES), jnp.float32),
        pltpu.VMEM_SHARED((N, LANES), jnp.float32),
        pltpu.SemaphoreType.DMA,
    ],
    compiler_params=_sc_params,
)


def _sc_agg_body(src_hbm, dst_hbm, ew_hbm, h_hbm, z_hbm,
                 acc_hbm,
                 src_v, dst_v, ew_v, rows_v, acc_sh, sem):
    cid = lax.axis_index("c")
    sid = lax.axis_index("s")

    r0 = sid * RS
    pltpu.sync_copy(z_hbm.at[pl.ds(r0, RS)], acc_sh.at[pl.ds(r0, RS)])

    @pl.when(sid == 0)
    def _():
        pltpu.sync_copy(z_hbm.at[pl.ds(RS * NSUB, TAIL)],
                        acc_sh.at[pl.ds(RS * NSUB, TAIL)])

    plsc.subcore_barrier()

    @pl.loop(sid, NCHUNK_SC, step=NSUB)
    def _(j):
        base = cid * E_PER_SC + j * EC
        pltpu.sync_copy(src_hbm.at[pl.ds(base, EC)], src_v)
        pltpu.sync_copy(dst_hbm.at[pl.ds(base, EC)], dst_v)
        pltpu.sync_copy(ew_hbm.at[pl.ds(base, EC)], ew_v)
        pltpu.async_copy(h_hbm.at[src_v], rows_v, sem).wait()

        @pl.loop(0, EC)
        def _(e):
            w16 = plsc.load_gather(ew_v, [jnp.full((LANES,), e, jnp.int32)])
            for k in range(HID // LANES):
                rows_v[e, pl.ds(k * LANES, LANES)] = (
                    rows_v[e, pl.ds(k * LANES, LANES)] * w16)

        pltpu.sync_copy(rows_v, acc_sh.at[dst_v], add=True)

    plsc.subcore_barrier()
    pltpu.sync_copy(acc_sh.at[pl.ds(r0, RS)],
                    acc_hbm.at[cid, pl.ds(r0, RS)])

    @pl.when(sid == 0)
    def _():
        pltpu.sync_copy(acc_sh.at[pl.ds(RS * NSUB, TAIL)],
                        acc_hbm.at[cid, pl.ds(RS * NSUB, TAIL)])


_sc_agg = pl.kernel(
    _sc_agg_body,
    out_type=jax.ShapeDtypeStruct((NSC, N, HID), jnp.float32),
    mesh=_mesh,
    scratch_types=[
        pltpu.VMEM((EC,), jnp.int32),
        pltpu.VMEM((EC,), jnp.int32),
        pltpu.VMEM((EC,), jnp.float32),
        pltpu.VMEM((EC, HID), jnp.float32),
        pltpu.VMEM_SHARED((N, HID), jnp.float32),
        pltpu.SemaphoreType.DMA,
    ],
    compiler_params=_sc_params,
)


def _tc1_body(degp_ref, nf_ref, xt_ref, xb_ref, tt_ref, w1_ref,
              dinv_ref, h2_ref):
    deg = degp_ref[0, :, 0:1] + degp_ref[1, :, 0:1] + 1.0
    dinv = lax.rsqrt(deg)
    dinv_ref[...] = dinv
    w1 = w1_ref[...]
    type_proj = jnp.dot(tt_ref[...], w1[NAME_D:NAME_D + TYPE_D, :],
                        preferred_element_type=jnp.float32)
    oh = jnp.where(
        lax.broadcasted_iota(jnp.int32, (N, TYPE_V), 1) == xt_ref[...],
        1.0, 0.0)
    xw = (jnp.dot(nf_ref[...], w1[:NAME_D, :],
                  preferred_element_type=jnp.float32)
          + jnp.dot(oh, type_proj, preferred_element_type=jnp.float32)
          + jnp.dot(xb_ref[...], w1[NAME_D + TYPE_D:, :],
                    preferred_element_type=jnp.float32))
    h2_ref[...] = dinv * xw


def _tc2_body(acc_ref, h2_ref, dinv_ref, b_ref, w2_ref, out_ref):
    dinv = dinv_ref[...]
    a = jnp.maximum(
        dinv * (acc_ref[0] + acc_ref[1] + h2_ref[...]) + b_ref[...], 0.0)
    out_ref[...] = dinv * jnp.dot(a, w2_ref[...],
                                  preferred_element_type=jnp.float32)


def _tc3_body(acc_ref, h2_ref, dinv_ref, b_ref, batch_ref, wc_ref, bc_ref,
              out_ref):
    a = jnp.maximum(
        dinv_ref[...] * (acc_ref[0] + acc_ref[1] + h2_ref[...]) + b_ref[...],
        0.0)
    oh = jnp.where(
        lax.broadcasted_iota(jnp.int32, (NGRAPH, N), 0) == batch_ref[...],
        1.0, 0.0)
    sums = jnp.dot(oh, a, preferred_element_type=jnp.float32)
    cnts = jnp.sum(oh, axis=1, keepdims=True)
    pooled = sums / jnp.maximum(cnts, 1.0)
    out_ref[...] = (jnp.dot(pooled, wc_ref[...],
                            preferred_element_type=jnp.float32) + bc_ref[...])


def kernel(x_names, x_types, x_behaviors, edge_index, edge_weight, batch,
           name_table, type_table, W1, b1, W2, b2, Wc, bc):
    src = edge_index[0].astype(jnp.int32)
    dst = edge_index[1].astype(jnp.int32)
    names = x_names.astype(jnp.int32)
    ew = edge_weight.astype(jnp.float32)
    xt = x_types.astype(jnp.int32).reshape(N, 1)
    batch2 = batch.astype(jnp.int32).reshape(1, N)
    z16 = jnp.zeros((N, LANES), jnp.float32)
    z128 = jnp.zeros((N, HID), jnp.float32)
    ncls = Wc.shape[1]

    degp, nfeat = _sc_prep(dst, ew, names, name_table.astype(jnp.float32),
                           z16)

    dinv, h2 = pl.pallas_call(
        _tc1_body,
        out_shape=(jax.ShapeDtypeStruct((N, 1), jnp.float32),
                   jax.ShapeDtypeStruct((N, HID), jnp.float32)),
    )(degp, nfeat, xt, x_behaviors.astype(jnp.float32),
      type_table.astype(jnp.float32), W1)

    acc1 = _sc_agg(src, dst, ew, h2, z128)

    h2b = pl.pallas_call(
        _tc2_body,
        out_shape=jax.ShapeDtypeStruct((N, HID), jnp.float32),
    )(acc1, h2, dinv, b1.reshape(1, HID), W2)

    acc2 = _sc_agg(src, dst, ew, h2b, z128)

    out = pl.pallas_call(
        _tc3_body,
        out_shape=jax.ShapeDtypeStruct((NGRAPH, ncls), jnp.float32),
    )(acc2, h2b, dinv, b2.reshape(1, HID), batch2, Wc, bc.reshape(1, ncls))
    return out


# speedup vs baseline: 14.9268x; 1.6555x over previous
import dataclasses

import jax
import jax.numpy as jnp
from jax import lax
from jax.experimental import pallas as pl
from jax.experimental.pallas import tpu as pltpu
from jax.experimental.pallas import tpu_sc as plsc

N = 10000
E = 320000
HID = 128
NGRAPH = 64
TYPE_V = 64
TYPE_D = 16
NAME_D = 64

NSC = 2
NSUB = 16
LANES = 16
NW = NSC * NSUB

EC = 80
TCH = E // EC
E_PER_SC = E // NSC
NCHUNK_SC = E_PER_SC // EC
T_SUB = NCHUNK_SC // NSUB
NBUF = 4
RS = 624
TAIL = N - RS * NSUB

NAMC = 80
NAME_CHUNKS = N // NAMC

_mesh = plsc.VectorSubcoreMesh(core_axis_name="c", subcore_axis_name="s")

_sc_params = pltpu.CompilerParams()
_fields = pltpu.CompilerParams.__dataclass_fields__
if "needs_layout_passes" in _fields:
    _sc_params = dataclasses.replace(_sc_params, needs_layout_passes=False)
if "use_tc_tiling_on_sc" in _fields:
    _sc_params = dataclasses.replace(_sc_params, use_tc_tiling_on_sc=False)


def _sc_prep_body(ep_hbm, names_hbm, table_hbm, z16_hbm,
                  degp_hbm, nfeat_hbm,
                  idx_v, nrow_v, ep_v, deg_rows, deg_sh, sem):
    cid = lax.axis_index("c")
    sid = lax.axis_index("s")
    wid = sid * NSC + cid

    @pl.loop(wid, NAME_CHUNKS, step=NW)
    def _(j):
        base = j * NAMC
        pltpu.sync_copy(names_hbm.at[pl.ds(base, NAMC)], idx_v)
        pltpu.async_copy(table_hbm.at[idx_v], nrow_v, sem).wait()
        pltpu.sync_copy(nrow_v, nfeat_hbm.at[pl.ds(base, NAMC)])

    r0 = sid * RS
    pltpu.sync_copy(z16_hbm.at[pl.ds(r0, RS)], deg_sh.at[pl.ds(r0, RS)])

    @pl.when(sid == 0)
    def _():
        pltpu.sync_copy(z16_hbm.at[pl.ds(RS * NSUB, TAIL)],
                        deg_sh.at[pl.ds(RS * NSUB, TAIL)])

    plsc.subcore_barrier()

    j0 = cid * NCHUNK_SC + sid

    @pl.loop(0, T_SUB)
    def _(t):
        pltpu.sync_copy(ep_hbm.at[j0 + t * NSUB], ep_v)

        @pl.loop(0, EC, unroll=4)
        def _(e):
            deg_rows[e, :] = plsc.bitcast(
                plsc.load_gather(ep_v.at[2],
                                 [jnp.full((LANES,), e, jnp.int32)]),
                jnp.float32)

        pltpu.sync_copy(deg_rows, deg_sh.at[ep_v.at[1]], add=True)

    plsc.subcore_barrier()
    pltpu.sync_copy(deg_sh.at[pl.ds(r0, RS)],
                    degp_hbm.at[cid, pl.ds(r0, RS)])

    @pl.when(sid == 0)
    def _():
        pltpu.sync_copy(deg_sh.at[pl.ds(RS * NSUB, TAIL)],
                        degp_hbm.at[cid, pl.ds(RS * NSUB, TAIL)])


_sc_prep = pl.kernel(
    _sc_prep_body,
    out_type=(jax.ShapeDtypeStruct((NSC, N, LANES), jnp.float32),
              jax.ShapeDtypeStruct((N, NAME_D), jnp.float32)),
    mesh=_mesh,
    scratch_types=[
        pltpu.VMEM((NAMC,), jnp.int32),
        pltpu.VMEM((NAMC, NAME_D), jnp.float32),
        pltpu.VMEM((3, EC), jnp.int32),
        pltpu.VMEM((EC, LANES), jnp.float32),
        pltpu.VMEM_SHARED((N, LANES), jnp.float32),
        pltpu.SemaphoreType.DMA,
    ],
    compiler_params=_sc_params,
)


def _sc_agg_body(ep_hbm, h_hbm, z_hbm,
                 acc_hbm,
                 ep_v, rows_v, acc_sh,
                 g0, g1, g2, g3, s0, s1, s2, s3):
    cid = lax.axis_index("c")
    sid = lax.axis_index("s")
    gsem = (g0, g1, g2, g3)
    ssem = (s0, s1, s2, s3)

    r0 = sid * RS
    pltpu.sync_copy(z_hbm.at[pl.ds(r0, RS)], acc_sh.at[pl.ds(r0, RS)])

    @pl.when(sid == 0)
    def _():
        pltpu.sync_copy(z_hbm.at[pl.ds(RS * NSUB, TAIL)],
                        acc_sh.at[pl.ds(RS * NSUB, TAIL)])

    plsc.subcore_barrier()

    j0 = cid * NCHUNK_SC + sid

    def fetch(t, b):
        pltpu.sync_copy(ep_hbm.at[j0 + t * NSUB], ep_v.at[b])
        pltpu.async_copy(h_hbm.at[ep_v.at[b, 0]], rows_v.at[b], gsem[b])

    def drain_scatter(b):
        pltpu.make_async_copy(rows_v.at[b], acc_sh.at[ep_v.at[b, 1]],
                              ssem[b]).wait()

    def process(t, b, wait_ssem):
        pltpu.make_async_copy(h_hbm.at[ep_v.at[b, 0]], rows_v.at[b],
                              gsem[b]).wait()

        @pl.loop(0, EC, unroll=4)
        def _(e):
            w16 = plsc.bitcast(
                plsc.load_gather(ep_v.at[b, 2],
                                 [jnp.full((LANES,), e, jnp.int32)]),
                jnp.float32)
            for k in range(HID // LANES):
                rows_v[b, e, pl.ds(k * LANES, LANES)] = (
                    rows_v[b, e, pl.ds(k * LANES, LANES)] * w16)

        pltpu.async_copy(rows_v.at[b], acc_sh.at[ep_v.at[b, 1]], ssem[b],
                         add=True)
        b2 = (b + 2) % NBUF
        if wait_ssem is None:
            fetch(t + 2, b2)
        elif wait_ssem:
            @pl.when(t + 2 < T_SUB)
            def _():
                drain_scatter(b2)
                fetch(t + 2, b2)

    fetch(0, 0)
    fetch(1, 1)
    process(0, 0, None)
    process(1, 1, None)
    process(2, 2, True)
    process(3, 3, True)

    @pl.loop(1, T_SUB // NBUF)
    def _(q):
        t = q * NBUF
        for p in range(NBUF):
            process(t + p, p, True)

    for t in range((T_SUB // NBUF) * NBUF, T_SUB):
        process(t, t % NBUF, True)

    for b in range(NBUF):
        drain_scatter(b)

    plsc.subcore_barrier()
    pltpu.sync_copy(acc_sh.at[pl.ds(r0, RS)],
                    acc_hbm.at[cid, pl.ds(r0, RS)])

    @pl.when(sid == 0)
    def _():
        pltpu.sync_copy(acc_sh.at[pl.ds(RS * NSUB, TAIL)],
                        acc_hbm.at[cid, pl.ds(RS * NSUB, TAIL)])


_sc_agg = pl.kernel(
    _sc_agg_body,
    out_type=jax.ShapeDtypeStruct((NSC, N, HID), jnp.float32),
    mesh=_mesh,
    scratch_types=[
        pltpu.VMEM((NBUF, 3, EC), jnp.int32),
        pltpu.VMEM((NBUF, EC, HID), jnp.float32),
        pltpu.VMEM_SHARED((N, HID), jnp.float32),
        pltpu.SemaphoreType.DMA,
        pltpu.SemaphoreType.DMA,
        pltpu.SemaphoreType.DMA,
        pltpu.SemaphoreType.DMA,
        pltpu.SemaphoreType.DMA,
        pltpu.SemaphoreType.DMA,
        pltpu.SemaphoreType.DMA,
        pltpu.SemaphoreType.DMA,
    ],
    compiler_params=_sc_params,
)


def _tc1_body(degp_ref, nf_ref, xt_ref, xb_ref, tt_ref, w1_ref,
              dinv_ref, h2_ref):
    deg = degp_ref[0, :, 0:1] + degp_ref[1, :, 0:1] + 1.0
    dinv = lax.rsqrt(deg)
    dinv_ref[...] = dinv
    w1 = w1_ref[...]
    type_proj = jnp.dot(tt_ref[...], w1[NAME_D:NAME_D + TYPE_D, :],
                        preferred_element_type=jnp.float32)
    oh = jnp.where(
        lax.broadcasted_iota(jnp.int32, (N, TYPE_V), 1) == xt_ref[...],
        1.0, 0.0)
    xw = (jnp.dot(nf_ref[...], w1[:NAME_D, :],
                  preferred_element_type=jnp.float32)
          + jnp.dot(oh, type_proj, preferred_element_type=jnp.float32)
          + jnp.dot(xb_ref[...], w1[NAME_D + TYPE_D:, :],
                    preferred_element_type=jnp.float32))
    h2_ref[...] = dinv * xw


def _tc2_body(acc_ref, h2_ref, dinv_ref, b_ref, w2_ref, out_ref):
    dinv = dinv_ref[...]
    a = jnp.maximum(
        dinv * (acc_ref[0] + acc_ref[1] + h2_ref[...]) + b_ref[...], 0.0)
    out_ref[...] = dinv * jnp.dot(a, w2_ref[...],
                                  preferred_element_type=jnp.float32)


def _tc3_body(acc_ref, h2_ref, dinv_ref, b_ref, batch_ref, wc_ref, bc_ref,
              out_ref):
    a = jnp.maximum(
        dinv_ref[...] * (acc_ref[0] + acc_ref[1] + h2_ref[...]) + b_ref[...],
        0.0)
    oh = jnp.where(
        lax.broadcasted_iota(jnp.int32, (NGRAPH, N), 0) == batch_ref[...],
        1.0, 0.0)
    sums = jnp.dot(oh, a, preferred_element_type=jnp.float32)
    cnts = jnp.sum(oh, axis=1, keepdims=True)
    pooled = sums / jnp.maximum(cnts, 1.0)
    out_ref[...] = (jnp.dot(pooled, wc_ref[...],
                            preferred_element_type=jnp.float32) + bc_ref[...])


def kernel(x_names, x_types, x_behaviors, edge_index, edge_weight, batch,
           name_table, type_table, W1, b1, W2, b2, Wc, bc):
    src = edge_index[0].astype(jnp.int32)
    dst = edge_index[1].astype(jnp.int32)
    names = x_names.astype(jnp.int32)
    ew_bits = lax.bitcast_convert_type(edge_weight.astype(jnp.float32),
                                       jnp.int32)
    epack = (jnp.stack([src, dst, ew_bits], axis=0)
             .reshape(3, TCH, EC).transpose(1, 0, 2))
    xt = x_types.astype(jnp.int32).reshape(N, 1)
    batch2 = batch.astype(jnp.int32).reshape(1, N)
    z16 = jnp.zeros((N, LANES), jnp.float32)
    z128 = jnp.zeros((N, HID), jnp.float32)
    ncls = Wc.shape[1]

    degp, nfeat = _sc_prep(epack, names, name_table.astype(jnp.float32),
                           z16)

    dinv, h2 = pl.pallas_call(
        _tc1_body,
        out_shape=(jax.ShapeDtypeStruct((N, 1), jnp.float32),
                   jax.ShapeDtypeStruct((N, HID), jnp.float32)),
    )(degp, nfeat, xt, x_behaviors.astype(jnp.float32),
      type_table.astype(jnp.float32), W1)

    acc1 = _sc_agg(epack, h2, z128)

    h2b = pl.pallas_call(
        _tc2_body,
        out_shape=jax.ShapeDtypeStruct((N, HID), jnp.float32),
    )(acc1, h2, dinv, b1.reshape(1, HID), W2)

    acc2 = _sc_agg(epack, h2b, z128)

    out = pl.pallas_call(
        _tc3_body,
        out_shape=jax.ShapeDtypeStruct((NGRAPH, ncls), jnp.float32),
    )(acc2, h2b, dinv, b2.reshape(1, HID), batch2, Wc, bc.reshape(1, ncls))
    return out
